# baseline (speedup 1.0000x reference)
import functools

import jax
import jax.numpy as jnp
from jax import lax
from jax.experimental import pallas as pl
from jax.experimental.pallas import tpu as pltpu
from jax.experimental.pallas import tpu_sc as plsc

B, T = 16384, 200
NC, NS, L = 2, 16, 16
NW = NC * NS
RPW = B // NW
PAIR_GROUPS = (RPW * 2) // L

_OUT = tuple(jax.ShapeDtypeStruct((B, T), jnp.float32) for _ in range(5))


@functools.partial(
    pl.kernel,
    out_type=_OUT,
    mesh=plsc.VectorSubcoreMesh(core_axis_name="c", subcore_axis_name="s"),
    scratch_types=[
        pltpu.VMEM((RPW, T), jnp.float32),
        pltpu.VMEM((RPW, 2), jnp.float32),
        pltpu.VMEM((RPW, 2), jnp.float32),
        pltpu.SemaphoreType.DMA,
        pltpu.SemaphoreType.DMA,
    ],
    compiler_params=pltpu.CompilerParams(
        use_tc_tiling_on_sc=False, needs_layout_passes=False
    ),
)
def _perturb(x_hbm, o0, o1, o2, o3, o4, buf, pin, pout, sem_in, sem_out):
    wid = lax.axis_index("s") * NC + lax.axis_index("c")
    base = wid * RPW
    rows = pl.ds(base, RPW)
    bulk_in = pltpu.async_copy(x_hbm.at[rows], buf, sem_in)
    pltpu.sync_copy(x_hbm.at[rows, pl.ds(0, 2)], pin)
    idx = lax.iota(jnp.int32, L)
    for g in range(PAIR_GROUPS):
        flat = idx + (g * L)
        r = lax.shift_right_logical(flat, 1)
        c = lax.bitwise_and(flat, 1)
        v = plsc.load_gather(pin, [r, lax.bitwise_xor(c, 1)])
        plsc.store_scatter(pout, [r, c], v)
    bulk_in.wait()
    writes = [
        pltpu.async_copy(buf, o.at[rows], sem_out) for o in (o0, o1, o2, o3, o4)
    ]
    for w in writes:
        w.wait()
    pltpu.sync_copy(pout, o1.at[rows, pl.ds(0, 2)])
    pltpu.sync_copy(pout, o3.at[rows, pl.ds(0, 2)])


def kernel(x):
    return _perturb(x)


# speedup vs baseline: 2.7666x; 2.7666x over previous
import functools

import jax
import jax.numpy as jnp
from jax import lax
from jax.experimental import pallas as pl
from jax.experimental.pallas import tpu as pltpu
from jax.experimental.pallas import tpu_sc as plsc

B, T = 16384, 200
NC, NS, L = 2, 16, 16
NW = NC * NS
RPW = B // NW
CHUNK = 256
NCHUNK = RPW // CHUNK
GROUPS = CHUNK // L


@functools.partial(
    pl.kernel,
    out_type=jax.ShapeDtypeStruct((B, T), jnp.float32),
    mesh=plsc.VectorSubcoreMesh(core_axis_name="c", subcore_axis_name="s"),
    scratch_types=[pltpu.VMEM((CHUNK, T), jnp.float32)],
    compiler_params=pltpu.CompilerParams(
        use_tc_tiling_on_sc=True, needs_layout_passes=False
    ),
)
def _swap01(x_hbm, y_hbm, buf):
    wid = lax.axis_index("s") * NC + lax.axis_index("c")
    lanes = lax.iota(jnp.int32, L)
    col0 = jnp.zeros((L,), jnp.int32)
    col1 = col0 + 1
    for ch in range(NCHUNK):
        base = wid * RPW + ch * CHUNK
        pltpu.sync_copy(x_hbm.at[pl.ds(base, CHUNK)], buf)
        for g in range(GROUPS):
            rows = lanes + (g * L)
            v0 = plsc.load_gather(buf, [rows, col0])
            v1 = plsc.load_gather(buf, [rows, col1])
            plsc.store_scatter(buf, [rows, col0], v1)
            plsc.store_scatter(buf, [rows, col1], v0)
        pltpu.sync_copy(buf, y_hbm.at[pl.ds(base, CHUNK)])


def kernel(x):
    y = _swap01(x)
    return (x, y, x, y, x)
